# baseline (speedup 1.0000x reference)
import jax
import jax.numpy as jnp
from jax import lax
from jax.experimental import pallas as pl
from jax.experimental.pallas import tpu as pltpu
from jax.experimental.pallas import tpu_sc as plsc

N_RAYS_S = 131072
N_COARSE_S = 64
N_FINE_S = 128
L = 16
NUM_CORES = 2
NUM_SUBCORES = 16
NUM_WORKERS = NUM_CORES * NUM_SUBCORES
RAYS_PER_WORKER = N_RAYS_S // NUM_WORKERS
CHUNK = 128
NUM_CHUNKS = RAYS_PER_WORKER // CHUNK
NGC = N_COARSE_S // L
NGF = N_FINE_S // L
CB_LEN = N_COARSE_S + L
CNT_LEN = 128
UNROLL = 2


def _tec_body(
    z_hbm, w_hbm, out_hbm,
    zbuf, wbuf, obuf0, obuf1, cbstore, slopestore, cntstore,
    zsem, wsem, osem0, osem1,
):
    wid = lax.axis_index("s") * NUM_CORES + lax.axis_index("c")
    base = wid * RAYS_PER_WORKER
    iota_f = lax.broadcasted_iota(jnp.int32, (L,), 0).astype(jnp.float32)
    zero16i = jnp.zeros((L,), jnp.int32)
    one16i = jnp.ones((L,), jnp.int32)
    zero16f = jnp.zeros((L,), jnp.float32)
    one16f = jnp.ones((L,), jnp.float32)
    idx15 = jnp.full((L,), L - 1, jnp.int32)
    iota_i = lax.broadcasted_iota(jnp.int32, (L,), 0)
    shift_idx = jnp.minimum(iota_i + 1, L - 1)
    lane15 = iota_i == (L - 1)

    @plsc.parallel_loop(0, CHUNK, 1, unroll=4)
    def _init(r):
        slopestore[r, pl.ds(N_COARSE_S, L)] = zero16f
        for g in range(CNT_LEN // L):
            cntstore[r, pl.ds(L * g, L)] = zero16i

    def in_copies(ci):
        row0 = base + ci * CHUNK
        return (
            pltpu.make_async_copy(z_hbm.at[pl.ds(row0, CHUNK)], zbuf, zsem),
            pltpu.make_async_copy(w_hbm.at[pl.ds(row0, CHUNK)], wbuf, wsem),
        )

    def stage_a_loop():
        @plsc.parallel_loop(0, CHUNK, 1, unroll=3)
        def stage_a(r):
            r16 = jnp.full((L,), r, jnp.int32)
            wv = [wbuf[r, pl.ds(L * g, L)] + 1e-5 for g in range(NGC)]
            cs = [plsc.cumsum(v) for v in wv]
            zv = [zbuf[r, pl.ds(L * g, L)] for g in range(NGC)]
            last = [c.at[idx15].get(mode="promise_in_bounds") for c in cs]
            run = [zero16f, last[0], last[0] + last[1], last[0] + last[1] + last[2]]
            inv = 127.0 / (run[3] + last[3])
            z63 = zv[NGC - 1].at[idx15].get(mode="promise_in_bounds")
            cbstore[r, pl.ds(N_COARSE_S, L)] = z63
            for g in range(NGC):
                d = wv[g] * inv
                cg = (cs[g] + run[g]) * inv
                rcp = 1.0 / jnp.where(d < 127e-5, jnp.float32(127.0), d)
                if g + 1 < NGC:
                    dz = zbuf[r, pl.ds(L * g + 1, L)] - zv[g]
                else:
                    zshift = zv[g].at[shift_idx].get(mode="promise_in_bounds")
                    dz = jnp.where(lane15, zero16f, zshift - zv[g])
                slope = dz * rcp
                slopestore[r, pl.ds(L * g, L)] = slope
                cbstore[r, pl.ds(L * g, L)] = zv[g] - (cg - d) * slope
                pgrid = 128 - (128.0 - cg).astype(jnp.int32)
                plsc.addupdate_scatter(
                    cntstore, [r16, jnp.minimum(pgrid, CNT_LEN - 1)], one16i
                )

    def stage_b_loop(ob):
        @plsc.parallel_loop(0, CHUNK, 1, unroll=UNROLL)
        def stage_b(r):
            r16 = jnp.full((L,), r, jnp.int32)
            carry = zero16i
            for g in range(NGF):
                local = plsc.cumsum(cntstore[r, pl.ds(L * g, L)])
                below = local + carry
                if g + 1 < NGF:
                    carry = carry + local.at[idx15].get(mode="promise_in_bounds")
                cntstore[r, pl.ds(L * g, L)] = zero16i
                a_v = plsc.load_gather(cbstore, [r16, below])
                slope = plsc.load_gather(slopestore, [r16, below])
                u = iota_f + jnp.float32(L * g)
                ob[r, pl.ds(L * g, L)] = a_v + u * slope

    def half(ci, i2, ob, osem):
        cz, cw = in_copies(ci)
        cz.wait()
        cw.wait()
        stage_a_loop()

        @pl.when(ci + 1 < NUM_CHUNKS)
        def _():
            nz, nw = in_copies(ci + 1)
            nz.start()
            nw.start()

        @pl.when(i2 > 0)
        def _():
            pltpu.make_async_copy(ob, out_hbm.at[pl.ds(base, CHUNK)], osem).wait()

        stage_b_loop(ob)
        row0 = base + ci * CHUNK
        pltpu.async_copy(ob, out_hbm.at[pl.ds(row0, CHUNK)], osem)

    cz0, cw0 = in_copies(0)
    cz0.start()
    cw0.start()

    def do_pair(i2, pair_carry):
        half(2 * i2, i2, obuf0, osem0)
        half(2 * i2 + 1, i2, obuf1, osem1)
        return pair_carry

    lax.fori_loop(0, NUM_CHUNKS // 2, do_pair, 0)
    pltpu.make_async_copy(obuf0, out_hbm.at[pl.ds(base, CHUNK)], osem0).wait()
    pltpu.make_async_copy(obuf1, out_hbm.at[pl.ds(base, CHUNK)], osem1).wait()


_sampler = pl.kernel(
    _tec_body,
    out_type=jax.ShapeDtypeStruct((N_RAYS_S, N_FINE_S), jnp.float32),
    mesh=plsc.VectorSubcoreMesh(core_axis_name="c", subcore_axis_name="s"),
    scratch_types=[
        pltpu.VMEM((CHUNK, N_COARSE_S), jnp.float32),
        pltpu.VMEM((CHUNK, N_COARSE_S), jnp.float32),
        pltpu.VMEM((CHUNK, N_FINE_S), jnp.float32),
        pltpu.VMEM((CHUNK, N_FINE_S), jnp.float32),
        pltpu.VMEM((CHUNK, CB_LEN), jnp.float32),
        pltpu.VMEM((CHUNK, CB_LEN), jnp.float32),
        pltpu.VMEM((CHUNK, CNT_LEN), jnp.int32),
        pltpu.SemaphoreType.DMA,
        pltpu.SemaphoreType.DMA,
        pltpu.SemaphoreType.DMA,
        pltpu.SemaphoreType.DMA,
    ],
    compiler_params=pltpu.CompilerParams(needs_layout_passes=False),
)


def kernel(z_vals, weights, num_samples):
    del num_samples
    return _sampler(z_vals, weights)


# speedup vs baseline: 1.1343x; 1.1343x over previous
import jax
import jax.numpy as jnp
from jax import lax
from jax.experimental import pallas as pl
from jax.experimental.pallas import tpu as pltpu
from jax.experimental.pallas import tpu_sc as plsc

N_RAYS_S = 131072
N_COARSE_S = 64
N_FINE_S = 128
L = 16
NUM_CORES = 2
NUM_SUBCORES = 16
NUM_WORKERS = NUM_CORES * NUM_SUBCORES
RAYS_PER_WORKER = N_RAYS_S // NUM_WORKERS
CHUNK = 128
NUM_CHUNKS = RAYS_PER_WORKER // CHUNK
NGC = N_COARSE_S // L
NGF = N_FINE_S // L
CB_LEN = N_COARSE_S + L
CNT_LEN = 128
UNROLL = 2


def _tec_body(
    z_hbm, w_hbm, out_hbm,
    zbuf, wbuf, obuf0, obuf1, cbstore, slopestore, cntstore,
    zsem, wsem, osem0, osem1,
):
    wid = lax.axis_index("s") * NUM_CORES + lax.axis_index("c")
    base = wid * RAYS_PER_WORKER
    iota_f = lax.broadcasted_iota(jnp.int32, (L,), 0).astype(jnp.float32)
    zero16i = jnp.zeros((L,), jnp.int32)
    one16i = jnp.ones((L,), jnp.int32)
    zero16f = jnp.zeros((L,), jnp.float32)
    one16f = jnp.ones((L,), jnp.float32)
    idx15 = jnp.full((L,), L - 1, jnp.int32)
    iota_i = lax.broadcasted_iota(jnp.int32, (L,), 0)
    shift_idx = jnp.minimum(iota_i + 1, L - 1)
    lane15 = iota_i == (L - 1)

    @plsc.parallel_loop(0, CHUNK, 1, unroll=4)
    def _init(r):
        slopestore[r, pl.ds(N_COARSE_S, L)] = zero16f
        for g in range(CNT_LEN // L):
            cntstore[r, pl.ds(L * g, L)] = zero16i

    def in_copies(ci):
        row0 = base + ci * CHUNK
        return (
            pltpu.make_async_copy(z_hbm.at[pl.ds(row0, CHUNK)], zbuf, zsem),
            pltpu.make_async_copy(w_hbm.at[pl.ds(row0, CHUNK)], wbuf, wsem),
        )

    def stage_a_loop():
        @plsc.parallel_loop(0, CHUNK, 1, unroll=UNROLL)
        def stage_a(r):
            r16 = jnp.full((L,), r, jnp.int32)
            wv = [wbuf[r, pl.ds(L * g, L)] + 1e-5 for g in range(NGC)]
            cs = [plsc.cumsum(v) for v in wv]
            zv = [zbuf[r, pl.ds(L * g, L)] for g in range(NGC)]
            last = [c.at[idx15].get(mode="promise_in_bounds") for c in cs]
            run = [zero16f, last[0], last[0] + last[1], last[0] + last[1] + last[2]]
            inv = 127.0 / (run[3] + last[3])
            z63 = zv[NGC - 1].at[idx15].get(mode="promise_in_bounds")
            cbstore[r, pl.ds(N_COARSE_S, L)] = z63
            for g in range(NGC):
                d = wv[g] * inv
                cg = (cs[g] + run[g]) * inv
                rcp = 1.0 / jnp.where(d < 127e-5, jnp.float32(127.0), d)
                if g + 1 < NGC:
                    dz = zbuf[r, pl.ds(L * g + 1, L)] - zv[g]
                else:
                    zshift = zv[g].at[shift_idx].get(mode="promise_in_bounds")
                    dz = jnp.where(lane15, zero16f, zshift - zv[g])
                slope = dz * rcp
                slopestore[r, pl.ds(L * g, L)] = slope
                cbstore[r, pl.ds(L * g, L)] = zv[g] - (cg - d) * slope
                pgrid = 128 - (128.0 - cg).astype(jnp.int32)
                plsc.addupdate_scatter(
                    cntstore, [r16, jnp.minimum(pgrid, CNT_LEN - 1)], one16i
                )

    def stage_b_loop(ob):
        @plsc.parallel_loop(0, CHUNK, 1, unroll=UNROLL)
        def stage_b(r):
            r16 = jnp.full((L,), r, jnp.int32)
            carry = zero16i
            for g in range(NGF):
                local = plsc.cumsum(cntstore[r, pl.ds(L * g, L)])
                below = local + carry
                if g + 1 < NGF:
                    carry = carry + local.at[idx15].get(mode="promise_in_bounds")
                cntstore[r, pl.ds(L * g, L)] = zero16i
                a_v = plsc.load_gather(cbstore, [r16, below])
                slope = plsc.load_gather(slopestore, [r16, below])
                u = iota_f + jnp.float32(L * g)
                ob[r, pl.ds(L * g, L)] = a_v + u * slope

    def half(ci, i2, ob, osem):
        cz, cw = in_copies(ci)
        cz.wait()
        cw.wait()
        stage_a_loop()

        @pl.when(ci + 1 < NUM_CHUNKS)
        def _():
            nz, nw = in_copies(ci + 1)
            nz.start()
            nw.start()

        @pl.when(i2 > 0)
        def _():
            pltpu.make_async_copy(ob, out_hbm.at[pl.ds(base, CHUNK)], osem).wait()

        stage_b_loop(ob)
        row0 = base + ci * CHUNK
        pltpu.async_copy(ob, out_hbm.at[pl.ds(row0, CHUNK)], osem)

    cz0, cw0 = in_copies(0)
    cz0.start()
    cw0.start()

    def do_pair(i2, pair_carry):
        half(2 * i2, i2, obuf0, osem0)
        half(2 * i2 + 1, i2, obuf1, osem1)
        return pair_carry

    lax.fori_loop(0, NUM_CHUNKS // 2, do_pair, 0)
    pltpu.make_async_copy(obuf0, out_hbm.at[pl.ds(base, CHUNK)], osem0).wait()
    pltpu.make_async_copy(obuf1, out_hbm.at[pl.ds(base, CHUNK)], osem1).wait()


_sampler = pl.kernel(
    _tec_body,
    out_type=jax.ShapeDtypeStruct((N_RAYS_S, N_FINE_S), jnp.float32),
    mesh=plsc.VectorSubcoreMesh(core_axis_name="c", subcore_axis_name="s"),
    scratch_types=[
        pltpu.VMEM((CHUNK, N_COARSE_S), jnp.float32),
        pltpu.VMEM((CHUNK, N_COARSE_S), jnp.float32),
        pltpu.VMEM((CHUNK, N_FINE_S), jnp.float32),
        pltpu.VMEM((CHUNK, N_FINE_S), jnp.float32),
        pltpu.VMEM((CHUNK, CB_LEN), jnp.float32),
        pltpu.VMEM((CHUNK, CB_LEN), jnp.float32),
        pltpu.VMEM((CHUNK, CNT_LEN), jnp.int32),
        pltpu.SemaphoreType.DMA,
        pltpu.SemaphoreType.DMA,
        pltpu.SemaphoreType.DMA,
        pltpu.SemaphoreType.DMA,
    ],
    compiler_params=pltpu.CompilerParams(needs_layout_passes=False),
)


def kernel(z_vals, weights, num_samples):
    del num_samples
    return _sampler(z_vals, weights)
